# baseline (speedup 1.0000x reference)
import functools

import jax
import jax.numpy as jnp
from jax import lax
from jax.experimental import pallas as pl
from jax.experimental.pallas import tpu as pltpu
from jax.experimental.pallas import tpu_sc as plsc

_NC = 2
_NS = 16
_NW = _NC * _NS
_IDXW = 128


def _sc_gather_mean(x, table, *, row_base, n_rows, patch):
    S = x.shape[1]
    n_patches = S // patch
    ct = n_patches * patch
    n_seg = ct // _IDXW
    ppr = _IDXW // patch
    orows = n_patches // ppr
    rw = n_rows // _NW
    n_chunks = rw
    assert n_rows % _NW == 0 and ct % _IDXW == 0 and n_patches % ppr == 0
    assert n_chunks >= 4 and n_chunks % 2 == 0
    assert rw == ppr or (rw < ppr and ppr % rw == 0)

    def body(x_hbm, table_hbm, feat_hbm, idx_v0, idx_v1, rows_v0, rows_v1,
             out_v0, out_v1, sem_i0, sem_i1, sem_g0, sem_g1, sem_o0, sem_o1):
        idx_v = (idx_v0, idx_v1)
        rows_v = (rows_v0, rows_v1)
        out_v = (out_v0, out_v1)
        sem_i = (sem_i0, sem_i1)
        sem_g = (sem_g0, sem_g1)
        sem_o = (sem_o0, sem_o1)
        wid = lax.axis_index("s") * _NC + lax.axis_index("c")

        def stage_idx(c, b):
            xrow = row_base + wid * rw + c
            pltpu.async_copy(x_hbm.at[pl.ds(xrow, 1)], idx_v[b], sem_i[b])

        def fire_gathers(c, b):
            pltpu.make_async_copy(
                x_hbm.at[pl.ds(0, 1)], idx_v[b], sem_i[b]).wait()
            for j in range(n_seg):
                pltpu.async_copy(
                    table_hbm.at[idx_v[b].at[0, pl.ds(j * _IDXW, _IDXW)]],
                    rows_v[b].at[pl.ds(j * _IDXW, _IDXW)],
                    sem_g[b])

        def wait_gathers(b):
            for j in range(n_seg):
                pltpu.make_async_copy(
                    table_hbm.at[idx_v[b].at[0, pl.ds(j * _IDXW, _IDXW)]],
                    rows_v[b].at[pl.ds(j * _IDXW, _IDXW)],
                    sem_g[b]).wait()

        def feat_dst(c):
            blk = (wid * rw) // ppr
            lane0 = (wid * rw) % ppr
            return feat_hbm.at[pl.ds(blk * n_patches, n_patches),
                               pl.ds((lane0 + c) * patch, patch)]

        def reduce_and_store(c, b, *, wait_out):
            if wait_out:
                pltpu.make_async_copy(
                    out_v[b], feat_dst(0), sem_o[b]).wait()

            def red(p, carry2):
                r0 = p * patch
                acc = rows_v[b][r0]
                for u in range(1, patch):
                    acc = acc + rows_v[b][r0 + u]
                out_v[b][p] = acc * (1.0 / patch)
                return carry2

            lax.fori_loop(0, n_patches, red, 0, unroll=4)
            pltpu.async_copy(out_v[b], feat_dst(c), sem_o[b])

        stage_idx(0, 0)
        stage_idx(1, 1)
        fire_gathers(0, 0)
        fire_gathers(1, 1)
        for c in (0, 1):
            b = c & 1
            wait_gathers(b)
            stage_idx(c + 2, b)
            reduce_and_store(c, b, wait_out=False)
            fire_gathers(c + 2, b)

        def main(i, carry):
            for b in range(2):
                c = 2 + 2 * i + b
                wait_gathers(b)
                stage_idx(c + 2, b)
                reduce_and_store(c, b, wait_out=True)
                fire_gathers(c + 2, b)
            return carry

        if n_chunks > 4:
            lax.fori_loop(0, (n_chunks - 4) // 2, main, 0, unroll=False)

        for c in (n_chunks - 2, n_chunks - 1):
            b = c & 1
            wait_gathers(b)
            reduce_and_store(c, b, wait_out=True)
        for b in range(2):
            pltpu.make_async_copy(
                out_v[b], feat_dst(0), sem_o[b]).wait()

    k = pl.kernel(
        body,
        out_type=jax.ShapeDtypeStruct((n_rows * S // _IDXW, _IDXW),
                                      jnp.float32),
        mesh=plsc.VectorSubcoreMesh(core_axis_name="c", subcore_axis_name="s"),
        scratch_types=[
            pltpu.VMEM((1, ct), jnp.int32),
            pltpu.VMEM((1, ct), jnp.int32),
            pltpu.VMEM((ct, patch), jnp.float32),
            pltpu.VMEM((ct, patch), jnp.float32),
            pltpu.VMEM((n_patches, patch), jnp.float32),
            pltpu.VMEM((n_patches, patch), jnp.float32),
            pltpu.SemaphoreType.DMA,
            pltpu.SemaphoreType.DMA,
            pltpu.SemaphoreType.DMA,
            pltpu.SemaphoreType.DMA,
            pltpu.SemaphoreType.DMA,
            pltpu.SemaphoreType.DMA,
        ],
        compiler_params=pltpu.CompilerParams(use_tc_tiling_on_sc=False),
    )
    return k(x, table)


def _tc_body(buf_ref, feat_ref, w_ref, b_ref, pos_ref, out_ref):
    f = feat_ref[...]
    bb = out_ref.shape[0]
    k = w_ref.shape[0]
    w = w_ref[...]
    addv = pos_ref[...] + b_ref[...]
    for t in range(bb):
        ft = f[:, t * k:(t + 1) * k]
        acc = lax.dot_general(
            ft, w, (((1,), (0,)), ((), ())),
            preferred_element_type=jnp.float32)
        out_ref[t] = acc + addv


def _tc_body_noalias(feat_ref, w_ref, b_ref, pos_ref, out_ref):
    _tc_body(None, feat_ref, w_ref, b_ref, pos_ref, out_ref)


def _tc_project_slice(buf, feat_s, W, b, pos2, *, block_off, b_total, bb):
    frows, fw = feat_s.shape
    P_ = pos2.shape[0]
    patch = W.shape[0]
    bs = frows * fw // (P_ * patch)
    D_ = W.shape[1]
    assert bb == fw // patch
    nb = bs // bb
    specs = [
        pl.BlockSpec((P_, fw), lambda i: (i, 0)),
        pl.BlockSpec((patch, D_), lambda i: (0, 0)),
        pl.BlockSpec((D_,), lambda i: (0,)),
        pl.BlockSpec((P_, D_), lambda i: (0, 0)),
    ]
    out_spec = pl.BlockSpec((bb, P_, D_),
                            lambda i, o=block_off: (o + i, 0, 0))
    out_shape = jax.ShapeDtypeStruct((b_total, P_, D_), jnp.float32)
    params = pltpu.CompilerParams(dimension_semantics=("arbitrary",))
    if buf is None:
        return pl.pallas_call(
            _tc_body_noalias, grid=(nb,), in_specs=specs,
            out_specs=out_spec, out_shape=out_shape,
            compiler_params=params,
        )(feat_s, W, b, pos2)
    return pl.pallas_call(
        _tc_body, grid=(nb,),
        in_specs=[pl.BlockSpec((1, 8, 128), lambda i: (0, 0, 0))] + specs,
        out_specs=out_spec, out_shape=out_shape,
        input_output_aliases={0: 0},
        compiler_params=params,
    )(buf, feat_s, W, b, pos2)


def kernel(x, table, W, b, pos_embed):
    B_, S_ = x.shape
    V_, patch = table.shape
    D_ = W.shape[1]
    n_patches = S_ // patch
    pos2 = pos_embed[0, :n_patches, :]

    sizes = [B_ // 4, B_ // 4, B_ // 4, B_ // 4]
    starts = [sum(sizes[:i]) for i in range(len(sizes))]
    feats = []
    x0 = lax.optimization_barrier(lax.slice_in_dim(x, 0, sizes[0], axis=0))
    feats.append(_sc_gather_mean(x0, table, row_base=0,
                                 n_rows=sizes[0], patch=patch))
    for s in range(1, len(sizes)):
        feats.append(_sc_gather_mean(x, table, row_base=starts[s],
                                     n_rows=sizes[s], patch=patch))
    buf = _tc_project_slice(None, feats[0], W, b, pos2,
                            block_off=0, b_total=B_, bb=8)
    for s in range(1, len(sizes)):
        feat_s, buf_dep = lax.optimization_barrier((feats[s], buf))
        buf = _tc_project_slice(buf_dep, feat_s, W, b, pos2,
                                block_off=starts[s] // 8, b_total=B_, bb=8)
    return buf


# speedup vs baseline: 1.0360x; 1.0360x over previous
import functools

import jax
import jax.numpy as jnp
from jax import lax
from jax.experimental import pallas as pl
from jax.experimental.pallas import tpu as pltpu
from jax.experimental.pallas import tpu_sc as plsc

_NC = 2
_NS = 16
_NW = _NC * _NS
_IDXW = 128


def _sc_gather_mean(x, table, *, row_base, n_rows, patch):
    S = x.shape[1]
    n_patches = S // patch
    ct = n_patches * patch
    n_seg = ct // _IDXW
    ppr = _IDXW // patch
    orows = n_patches // ppr
    rw = n_rows // _NW
    n_chunks = rw
    assert n_rows % _NW == 0 and ct % _IDXW == 0 and n_patches % ppr == 0
    assert n_chunks >= 4 and n_chunks % 2 == 0
    assert rw == ppr or (rw < ppr and ppr % rw == 0)

    def body(x_hbm, table_hbm, feat_hbm, idx_v0, idx_v1, rows_v0, rows_v1,
             out_v0, out_v1, sem_i0, sem_i1, sem_g0, sem_g1, sem_o0, sem_o1):
        idx_v = (idx_v0, idx_v1)
        rows_v = (rows_v0, rows_v1)
        out_v = (out_v0, out_v1)
        sem_i = (sem_i0, sem_i1)
        sem_g = (sem_g0, sem_g1)
        sem_o = (sem_o0, sem_o1)
        wid = lax.axis_index("s") * _NC + lax.axis_index("c")

        def stage_idx(c, b):
            xrow = row_base + wid * rw + c
            pltpu.async_copy(x_hbm.at[pl.ds(xrow, 1)], idx_v[b], sem_i[b])

        def fire_gathers(c, b):
            pltpu.make_async_copy(
                x_hbm.at[pl.ds(0, 1)], idx_v[b], sem_i[b]).wait()
            for j in range(n_seg):
                pltpu.async_copy(
                    table_hbm.at[idx_v[b].at[0, pl.ds(j * _IDXW, _IDXW)]],
                    rows_v[b].at[pl.ds(j * _IDXW, _IDXW)],
                    sem_g[b])

        def wait_gathers(b):
            for j in range(n_seg):
                pltpu.make_async_copy(
                    table_hbm.at[idx_v[b].at[0, pl.ds(j * _IDXW, _IDXW)]],
                    rows_v[b].at[pl.ds(j * _IDXW, _IDXW)],
                    sem_g[b]).wait()

        def feat_dst(c):
            blk = (wid * rw) // ppr
            lane0 = (wid * rw) % ppr
            return feat_hbm.at[pl.ds(blk * n_patches, n_patches),
                               pl.ds((lane0 + c) * patch, patch)]

        def reduce_and_store(c, b, *, wait_out):
            if wait_out:
                pltpu.make_async_copy(
                    out_v[b], feat_dst(0), sem_o[b]).wait()

            def red(p, carry2):
                r0 = p * patch
                acc = rows_v[b][r0]
                for u in range(1, patch):
                    acc = acc + rows_v[b][r0 + u]
                out_v[b][p] = acc * (1.0 / patch)
                return carry2

            lax.fori_loop(0, n_patches, red, 0, unroll=4)
            pltpu.async_copy(out_v[b], feat_dst(c), sem_o[b])

        stage_idx(0, 0)
        stage_idx(1, 1)
        fire_gathers(0, 0)
        fire_gathers(1, 1)
        for c in (0, 1):
            b = c & 1
            wait_gathers(b)
            stage_idx(c + 2, b)
            reduce_and_store(c, b, wait_out=False)
            fire_gathers(c + 2, b)

        def main(i, carry):
            for b in range(2):
                c = 2 + 2 * i + b
                wait_gathers(b)
                stage_idx(c + 2, b)
                reduce_and_store(c, b, wait_out=True)
                fire_gathers(c + 2, b)
            return carry

        if n_chunks > 4:
            lax.fori_loop(0, (n_chunks - 4) // 2, main, 0, unroll=False)

        for c in (n_chunks - 2, n_chunks - 1):
            b = c & 1
            wait_gathers(b)
            reduce_and_store(c, b, wait_out=True)
        for b in range(2):
            pltpu.make_async_copy(
                out_v[b], feat_dst(0), sem_o[b]).wait()

    k = pl.kernel(
        body,
        out_type=jax.ShapeDtypeStruct((n_rows * S // _IDXW, _IDXW),
                                      jnp.float32),
        mesh=plsc.VectorSubcoreMesh(core_axis_name="c", subcore_axis_name="s"),
        scratch_types=[
            pltpu.VMEM((1, ct), jnp.int32),
            pltpu.VMEM((1, ct), jnp.int32),
            pltpu.VMEM((ct, patch), jnp.float32),
            pltpu.VMEM((ct, patch), jnp.float32),
            pltpu.VMEM((n_patches, patch), jnp.float32),
            pltpu.VMEM((n_patches, patch), jnp.float32),
            pltpu.SemaphoreType.DMA,
            pltpu.SemaphoreType.DMA,
            pltpu.SemaphoreType.DMA,
            pltpu.SemaphoreType.DMA,
            pltpu.SemaphoreType.DMA,
            pltpu.SemaphoreType.DMA,
        ],
        compiler_params=pltpu.CompilerParams(use_tc_tiling_on_sc=False),
    )
    return k(x, table)


def _tc_body(buf_ref, feat_ref, w_ref, b_ref, pos_ref, out_ref):
    f = feat_ref[...]
    bb = out_ref.shape[0]
    k = w_ref.shape[0]
    w = w_ref[...]
    addv = pos_ref[...] + b_ref[...]
    for t in range(bb):
        ft = f[:, t * k:(t + 1) * k]
        acc = lax.dot_general(
            ft, w, (((1,), (0,)), ((), ())),
            preferred_element_type=jnp.float32)
        out_ref[t] = acc + addv


def _tc_body_noalias(feat_ref, w_ref, b_ref, pos_ref, out_ref):
    _tc_body(None, feat_ref, w_ref, b_ref, pos_ref, out_ref)


def _tc_project_slice(buf, feat_s, W, b, pos2, *, block_off, b_total, bb):
    frows, fw = feat_s.shape
    P_ = pos2.shape[0]
    patch = W.shape[0]
    bs = frows * fw // (P_ * patch)
    D_ = W.shape[1]
    assert bb == fw // patch
    nb = bs // bb
    specs = [
        pl.BlockSpec((P_, fw), lambda i: (i, 0)),
        pl.BlockSpec((patch, D_), lambda i: (0, 0)),
        pl.BlockSpec((D_,), lambda i: (0,)),
        pl.BlockSpec((P_, D_), lambda i: (0, 0)),
    ]
    out_spec = pl.BlockSpec((bb, P_, D_),
                            lambda i, o=block_off: (o + i, 0, 0))
    out_shape = jax.ShapeDtypeStruct((b_total, P_, D_), jnp.float32)
    params = pltpu.CompilerParams(dimension_semantics=("arbitrary",))
    if buf is None:
        return pl.pallas_call(
            _tc_body_noalias, grid=(nb,), in_specs=specs,
            out_specs=out_spec, out_shape=out_shape,
            compiler_params=params,
        )(feat_s, W, b, pos2)
    return pl.pallas_call(
        _tc_body, grid=(nb,),
        in_specs=[pl.BlockSpec((1, 8, 128), lambda i: (0, 0, 0))] + specs,
        out_specs=out_spec, out_shape=out_shape,
        input_output_aliases={0: 0},
        compiler_params=params,
    )(buf, feat_s, W, b, pos2)


def kernel(x, table, W, b, pos_embed):
    B_, S_ = x.shape
    V_, patch = table.shape
    D_ = W.shape[1]
    n_patches = S_ // patch
    pos2 = pos_embed[0, :n_patches, :]

    sizes = [B_ // 4, B_ // 4, B_ // 4, B_ // 4]
    starts = [sum(sizes[:i]) for i in range(len(sizes))]
    feats = []
    for s, (st, sz) in enumerate(zip(starts, sizes)):
        x_s = lax.optimization_barrier(
            lax.slice_in_dim(x, st, st + sz, axis=0))
        feats.append(_sc_gather_mean(x_s, table, row_base=0,
                                     n_rows=sz, patch=patch))
    buf = _tc_project_slice(None, feats[0], W, b, pos2,
                            block_off=0, b_total=B_, bb=8)
    for s in range(1, len(sizes)):
        feat_s, buf_dep = lax.optimization_barrier((feats[s], buf))
        buf = _tc_project_slice(buf_dep, feat_s, W, b, pos2,
                                block_off=starts[s] // 8, b_total=B_, bb=8)
    return buf


# speedup vs baseline: 1.0372x; 1.0012x over previous
import functools

import jax
import jax.numpy as jnp
from jax import lax
from jax.experimental import pallas as pl
from jax.experimental.pallas import tpu as pltpu
from jax.experimental.pallas import tpu_sc as plsc

_NC = 2
_NS = 16
_NW = _NC * _NS
_IDXW = 128


def _sc_gather_mean(x, table, *, row_base, n_rows, patch):
    S = x.shape[1]
    n_patches = S // patch
    ct = n_patches * patch
    n_seg = ct // _IDXW
    ppr = _IDXW // patch
    orows = n_patches // ppr
    rw = n_rows // _NW
    n_chunks = rw
    assert n_rows % _NW == 0 and ct % _IDXW == 0 and n_patches % ppr == 0
    assert n_chunks >= 4 and n_chunks % 2 == 0

    def body(x_hbm, table_hbm, feat_hbm, idx_v0, idx_v1, rows_v0, rows_v1,
             out_v0, out_v1, sem_i0, sem_i1, sem_g0, sem_g1, sem_o0, sem_o1):
        idx_v = (idx_v0, idx_v1)
        rows_v = (rows_v0, rows_v1)
        out_v = (out_v0, out_v1)
        sem_i = (sem_i0, sem_i1)
        sem_g = (sem_g0, sem_g1)
        sem_o = (sem_o0, sem_o1)
        wid = lax.axis_index("s") * _NC + lax.axis_index("c")

        def stage_idx(c, b):
            xrow = row_base + wid * rw + c
            pltpu.async_copy(x_hbm.at[pl.ds(xrow, 1)], idx_v[b], sem_i[b])

        def fire_gathers(c, b):
            pltpu.make_async_copy(
                x_hbm.at[pl.ds(0, 1)], idx_v[b], sem_i[b]).wait()
            for j in range(n_seg):
                pltpu.async_copy(
                    table_hbm.at[idx_v[b].at[0, pl.ds(j * _IDXW, _IDXW)]],
                    rows_v[b].at[pl.ds(j * _IDXW, _IDXW)],
                    sem_g[b])

        def wait_gathers(b):
            for j in range(n_seg):
                pltpu.make_async_copy(
                    table_hbm.at[idx_v[b].at[0, pl.ds(j * _IDXW, _IDXW)]],
                    rows_v[b].at[pl.ds(j * _IDXW, _IDXW)],
                    sem_g[b]).wait()

        def feat_dst(c):
            return feat_hbm.at[pl.ds(wid * n_patches, n_patches),
                               pl.ds(c * patch, patch)]

        def reduce_and_store(c, b, *, wait_out):
            if wait_out:
                pltpu.make_async_copy(
                    out_v[b], feat_dst(0), sem_o[b]).wait()

            def red(p, carry2):
                r0 = p * patch
                acc = rows_v[b][r0]
                for u in range(1, patch):
                    acc = acc + rows_v[b][r0 + u]
                out_v[b][p] = acc * (1.0 / patch)
                return carry2

            lax.fori_loop(0, n_patches, red, 0, unroll=4)
            pltpu.async_copy(out_v[b], feat_dst(c), sem_o[b])

        stage_idx(0, 0)
        stage_idx(1, 1)
        fire_gathers(0, 0)
        fire_gathers(1, 1)
        for c in (0, 1):
            b = c & 1
            wait_gathers(b)
            stage_idx(c + 2, b)
            reduce_and_store(c, b, wait_out=False)
            fire_gathers(c + 2, b)

        def main(i, carry):
            for b in range(2):
                c = 2 + 2 * i + b
                wait_gathers(b)
                stage_idx(c + 2, b)
                reduce_and_store(c, b, wait_out=True)
                fire_gathers(c + 2, b)
            return carry

        if n_chunks > 4:
            lax.fori_loop(0, (n_chunks - 4) // 2, main, 0, unroll=False)

        for c in (n_chunks - 2, n_chunks - 1):
            b = c & 1
            wait_gathers(b)
            reduce_and_store(c, b, wait_out=True)
        for b in range(2):
            pltpu.make_async_copy(
                out_v[b], feat_dst(0), sem_o[b]).wait()

    k = pl.kernel(
        body,
        out_type=jax.ShapeDtypeStruct((n_rows * S // _IDXW, _IDXW),
                                      jnp.float32),
        mesh=plsc.VectorSubcoreMesh(core_axis_name="c", subcore_axis_name="s"),
        scratch_types=[
            pltpu.VMEM((1, ct), jnp.int32),
            pltpu.VMEM((1, ct), jnp.int32),
            pltpu.VMEM((ct, patch), jnp.float32),
            pltpu.VMEM((ct, patch), jnp.float32),
            pltpu.VMEM((n_patches, patch), jnp.float32),
            pltpu.VMEM((n_patches, patch), jnp.float32),
            pltpu.SemaphoreType.DMA,
            pltpu.SemaphoreType.DMA,
            pltpu.SemaphoreType.DMA,
            pltpu.SemaphoreType.DMA,
            pltpu.SemaphoreType.DMA,
            pltpu.SemaphoreType.DMA,
        ],
        compiler_params=pltpu.CompilerParams(use_tc_tiling_on_sc=False),
    )
    return k(x, table)


def _tc_body(buf_ref, feat_ref, w_ref, b_ref, pos_ref, out_ref):
    f = feat_ref[...]
    bb = out_ref.shape[0]
    k = w_ref.shape[0]
    w = w_ref[...]
    addv = pos_ref[...] + b_ref[...]
    for t in range(bb):
        ft = f[:, t * k:(t + 1) * k]
        acc = lax.dot_general(
            ft, w, (((1,), (0,)), ((), ())),
            preferred_element_type=jnp.float32)
        out_ref[t] = acc + addv


def _tc_body_noalias(feat_ref, w_ref, b_ref, pos_ref, out_ref):
    _tc_body(None, feat_ref, w_ref, b_ref, pos_ref, out_ref)


def _tc_project_slice(buf, feat_s, W, b, pos2, *, s, b_total, bb):
    frows, fw = feat_s.shape
    P_ = pos2.shape[0]
    patch = W.shape[0]
    bs = frows * fw // (P_ * patch)
    D_ = W.shape[1]
    assert bb == fw // patch
    nb = bs // bb
    specs = [
        pl.BlockSpec((P_, fw), lambda i: (i, 0)),
        pl.BlockSpec((patch, D_), lambda i: (0, 0)),
        pl.BlockSpec((D_,), lambda i: (0,)),
        pl.BlockSpec((P_, D_), lambda i: (0, 0)),
    ]
    out_spec = pl.BlockSpec((bb, P_, D_), lambda i, s=s: (s * nb + i, 0, 0))
    out_shape = jax.ShapeDtypeStruct((b_total, P_, D_), jnp.float32)
    params = pltpu.CompilerParams(dimension_semantics=("arbitrary",))
    if buf is None:
        return pl.pallas_call(
            _tc_body_noalias, grid=(nb,), in_specs=specs,
            out_specs=out_spec, out_shape=out_shape,
            compiler_params=params,
        )(feat_s, W, b, pos2)
    return pl.pallas_call(
        _tc_body, grid=(nb,),
        in_specs=[pl.BlockSpec((1, 8, 128), lambda i: (0, 0, 0))] + specs,
        out_specs=out_spec, out_shape=out_shape,
        input_output_aliases={0: 0},
        compiler_params=params,
    )(buf, feat_s, W, b, pos2)


def kernel(x, table, W, b, pos_embed):
    B_, S_ = x.shape
    V_, patch = table.shape
    D_ = W.shape[1]
    n_patches = S_ // patch
    pos2 = pos_embed[0, :n_patches, :]

    nsplit = 4
    bs = B_ // nsplit
    feats = []
    for s in range(nsplit):
        feats.append(_sc_gather_mean(x, table, row_base=s * bs,
                                     n_rows=bs, patch=patch))
    buf = _tc_project_slice(None, feats[0], W, b, pos2,
                            s=0, b_total=B_, bb=8)
    for s in range(1, nsplit):
        feat_s, buf_dep = lax.optimization_barrier((feats[s], buf))
        buf = _tc_project_slice(buf_dep, feat_s, W, b, pos2,
                                s=s, b_total=B_, bb=8)
    return buf
